# baseline (speedup 1.0000x reference)
import jax
import jax.numpy as jnp
from jax import lax
from jax.experimental import pallas as pl
from jax.experimental.pallas import tpu as pltpu
from jax.experimental.pallas import tpu_sc as plsc

N = 10000
D = 128
H = 128
O = 128
E = 320000
W = 144
NC = 2
NS = 16
NW = NC * NS
EPT_REAL = E // NW
K = 64
G = 3
NSETS = 4
SLOTS = NSETS * G
EPT = 10752
CH = EPT // K
BN = 1000
NP = 10112
ROWS_PT = NP // NS

_SC_PARAMS = dict(
    compiler_params=pltpu.CompilerParams(
        needs_layout_passes=False, use_tc_tiling_on_sc=False))


def _proj_body(x_ref, ws_ref, wd_ref, ats_ref, atd_ref,
               xsp_ref, as_ref, ad_ref):
    xb = x_ref[...]
    xs = jnp.dot(xb, ws_ref[...], preferred_element_type=jnp.float32,
                 precision=lax.Precision.HIGHEST)
    xd = jnp.dot(xb, wd_ref[...], preferred_element_type=jnp.float32,
                 precision=lax.Precision.HIGHEST)
    as_ref[...] = jnp.sum(xs * ats_ref[...], axis=1, keepdims=True)
    ad_ref[...] = jnp.sum(xd * atd_ref[...], axis=1, keepdims=True)
    ones = jnp.ones((BN, 1), jnp.float32)
    zeros = jnp.zeros((BN, W - H - 1), jnp.float32)
    xsp_ref[...] = jnp.concatenate([xs, ones, zeros], axis=1)


def _project(x, W_src, W_dst, att_src, att_dst):
    return pl.pallas_call(
        _proj_body,
        grid=(N // BN,),
        in_specs=[
            pl.BlockSpec((BN, D), lambda i: (i, 0)),
            pl.BlockSpec((D, H), lambda i: (0, 0)),
            pl.BlockSpec((D, H), lambda i: (0, 0)),
            pl.BlockSpec((1, H), lambda i: (0, 0)),
            pl.BlockSpec((1, H), lambda i: (0, 0)),
        ],
        out_specs=[
            pl.BlockSpec((BN, W), lambda i: (i, 0)),
            pl.BlockSpec((BN, 1), lambda i: (i, 0)),
            pl.BlockSpec((BN, 1), lambda i: (i, 0)),
        ],
        out_shape=[
            jax.ShapeDtypeStruct((N, W), jnp.float32),
            jax.ShapeDtypeStruct((N, 1), jnp.float32),
            jax.ShapeDtypeStruct((N, 1), jnp.float32),
        ],
    )(x, W_src, W_dst, att_src.reshape(1, H), att_dst.reshape(1, H))


def _weights_body(src_hbm, dst_hbm, as_hbm, ad_hbm, e_hbm,
                  as_v, ad_v, src_sl, dst_sl, e_sl):
    c = lax.axis_index("c")
    s = lax.axis_index("s")
    wid = s * NC + c
    row0 = wid * CH

    pltpu.sync_copy(as_hbm, as_v)
    pltpu.sync_copy(ad_hbm, ad_v)
    pltpu.sync_copy(src_hbm.at[pl.ds(row0, CH)], src_sl)
    pltpu.sync_copy(dst_hbm.at[pl.ds(row0, CH)], dst_sl)

    def _row(r, carry):
        for half in range(K // 16):
            s16 = src_sl[r, pl.ds(half * 16, 16)]
            d16 = dst_sl[r, pl.ds(half * 16, 16)]
            al = plsc.load_gather(as_v, [s16]) + plsc.load_gather(ad_v, [d16])
            al = jnp.where(al >= 0.0, al, al * jnp.float32(0.2))
            ex = jnp.exp(al)
            lid = r * K + half * 16 + lax.iota(jnp.int32, 16)
            e_sl[r, pl.ds(half * 16, 16)] = jnp.where(
                lid < EPT_REAL, ex, jnp.float32(0.0))
        return carry

    lax.fori_loop(0, CH, _row, 0)
    pltpu.sync_copy(e_sl, e_hbm.at[pl.ds(row0, CH)])


def _edge_weights(src2, dst2, a_s, a_d):
    mesh = plsc.VectorSubcoreMesh(core_axis_name="c", subcore_axis_name="s")
    f = pl.kernel(
        _weights_body,
        mesh=mesh,
        out_type=jax.ShapeDtypeStruct((NW * CH, K), jnp.float32),
        scratch_types=[
            pltpu.VMEM((N,), jnp.float32),
            pltpu.VMEM((N,), jnp.float32),
            pltpu.VMEM((CH, K), jnp.int32),
            pltpu.VMEM((CH, K), jnp.int32),
            pltpu.VMEM((CH, K), jnp.float32),
        ],
        **_SC_PARAMS,
    )
    return f(src2, dst2, a_s, a_d)


def _msg_body(xsp_hbm, src_hbm, dst_hbm, e_hbm, out_hbm,
              ss0, ss1, ss2, ss3, ds0, ds1, ds2, ds3,
              es0, es1, es2, es3,
              r0, r1, r2, ebc, h_sh,
              g0, g1, g2, q0, q1, q2, l0, l1, l2, l3):
    srcs = (ss0, ss1, ss2, ss3)
    dsts = (ds0, ds1, ds2, ds3)
    es = (es0, es1, es2, es3)
    rows = (r0, r1, r2)
    gsem = (g0, g1, g2)
    ssem = (q0, q1, q2)
    lsem = (l0, l1, l2, l3)

    c = lax.axis_index("c")
    s = lax.axis_index("s")
    wid = s * NC + c
    row0 = wid * CH

    def _zero_row(k, carry):
        for m in range(W // 16):
            r0[k, pl.ds(m * 16, 16)] = jnp.zeros((16,), jnp.float32)
        return carry
    lax.fori_loop(0, K, _zero_row, 0)
    for i in range(ROWS_PT // K):
        pltpu.sync_copy(r0, h_sh.at[pl.ds(s * ROWS_PT + i * K, K)])
    _rem = ROWS_PT % K
    if _rem:
        pltpu.sync_copy(
            r0.at[pl.ds(0, _rem)],
            h_sh.at[pl.ds(s * ROWS_PT + (ROWS_PT // K) * K, _rem)])
    plsc.subcore_barrier()

    for grp in range(2):
        pltpu.sync_copy(src_hbm.at[pl.ds(row0 + grp * G, G)], srcs[grp])
        pltpu.sync_copy(dst_hbm.at[pl.ds(row0 + grp * G, G)], dsts[grp])
        pltpu.sync_copy(e_hbm.at[pl.ds(row0 + grp * G, G)], es[grp])
    pltpu.async_copy(xsp_hbm.at[ss0.at[0]], r0, g0)
    pltpu.async_copy(xsp_hbm.at[ss0.at[1]], r1, g1)

    def _outer(i, carry):
        for b in range(SLOTS):
            ci = i * SLOTS + b
            p = b % 3
            set_c = b // 3
            if b % 3 == 0:
                set_s = (set_c + 2) % NSETS
                @pl.when(ci + 2 * G < CH)
                def _():
                    gro = row0 + ci + 2 * G
                    pltpu.async_copy(src_hbm.at[pl.ds(gro, G)],
                                     srcs[set_s], lsem[set_s])
                    pltpu.async_copy(dst_hbm.at[pl.ds(gro, G)],
                                     dsts[set_s], lsem[set_s])
                    pltpu.async_copy(e_hbm.at[pl.ds(gro, G)],
                                     es[set_s], lsem[set_s])
            if b % 3 == 1:
                set_w = ((b + 2) // 3) % NSETS
                @pl.when((ci >= 4) & (ci + 2 < CH))
                def _():
                    gro = row0 + ci + 2
                    pltpu.make_async_copy(src_hbm.at[pl.ds(gro, G)],
                                          srcs[set_w], lsem[set_w]).wait()
                    pltpu.make_async_copy(dst_hbm.at[pl.ds(gro, G)],
                                          dsts[set_w], lsem[set_w]).wait()
                    pltpu.make_async_copy(e_hbm.at[pl.ds(gro, G)],
                                          es[set_w], lsem[set_w]).wait()
            pb = (b - 1) % SLOTS
            set_d, row_d, q = (pb // 3) % NSETS, pb % 3, pb % 3
            @pl.when(ci >= 1)
            def _():
                pltpu.make_async_copy(
                    rows[q], h_sh.at[dsts[set_d].at[row_d]], ssem[q]).wait()
            nb = b + 2
            set_l, row_l, ql = (nb // 3) % NSETS, nb % 3, nb % 3
            @pl.when(ci + 2 < CH)
            def _():
                pltpu.async_copy(xsp_hbm.at[srcs[set_l].at[row_l]],
                                 rows[ql], gsem[ql])
            for j in range(K // 16):
                e16 = es[set_c][p, pl.ds(j * 16, 16)]
                for t in range(16):
                    ebc[j * 16 + t, pl.ds(0, 16)] = jnp.full(
                        (16,), e16[t], jnp.float32)
            pltpu.make_async_copy(xsp_hbm.at[srcs[set_c].at[p]],
                                  rows[p], gsem[p]).wait()
            def _scale(kk, carry2, _ro=rows[p]):
                for u in range(8):
                    k = kk * 8 + u
                    ek = ebc[k, pl.ds(0, 16)]
                    for m in range(W // 16):
                        _ro[k, pl.ds(m * 16, 16)] = (
                            _ro[k, pl.ds(m * 16, 16)] * ek)
                return carry2
            lax.fori_loop(0, K // 8, _scale, 0)
            pltpu.async_copy(rows[p], h_sh.at[dsts[set_c].at[p]],
                             ssem[p], add=True)
        return carry

    lax.fori_loop(0, CH // SLOTS, _outer, 0)

    pltpu.make_async_copy(
        rows[(CH - 1) % 3],
        h_sh.at[dsts[((CH - 1) // 3) % NSETS].at[(CH - 1) % 3]],
        ssem[(CH - 1) % 3]).wait()

    plsc.subcore_barrier()
    for i in range(ROWS_PT // K):
        pltpu.sync_copy(h_sh.at[pl.ds(s * ROWS_PT + i * K, K)],
                        out_hbm.at[c, pl.ds(s * ROWS_PT + i * K, K)])
    if _rem:
        off_r = s * ROWS_PT + (ROWS_PT // K) * K
        pltpu.sync_copy(h_sh.at[pl.ds(off_r, _rem)],
                        out_hbm.at[c, pl.ds(off_r, _rem)])


def _edge_pass(xsp, src2, dst2, e2):
    mesh = plsc.VectorSubcoreMesh(core_axis_name="c", subcore_axis_name="s")
    f = pl.kernel(
        _msg_body,
        mesh=mesh,
        out_type=jax.ShapeDtypeStruct((NC, NP, W), jnp.float32),
        scratch_types=(
            [pltpu.VMEM((G, K), jnp.int32)] * 4
            + [pltpu.VMEM((G, K), jnp.int32)] * 4
            + [pltpu.VMEM((G, K), jnp.float32)] * 4
            + [pltpu.VMEM((K, W), jnp.float32)] * 3
            + [pltpu.VMEM((K, 16), jnp.float32)]
            + [pltpu.VMEM_SHARED((NP, W), jnp.float32)]
            + [pltpu.SemaphoreType.DMA] * 10
        ),
        **_SC_PARAMS,
    )
    return f(xsp, src2, dst2, e2)


def _out_body(hp_ref, bias_ref, wl_ref, bl_ref, o_ref):
    num = hp_ref[0, :, 0:H] + hp_ref[1, :, 0:H]
    den = hp_ref[0, :, H:H + 1] + hp_ref[1, :, H:H + 1]
    h = num / (den + jnp.float32(1e-16)) + bias_ref[...]
    h = jnp.maximum(h, 0.0)
    o_ref[...] = jnp.dot(h, wl_ref[...], preferred_element_type=jnp.float32,
                         precision=lax.Precision.HIGHEST) + bl_ref[...]


def _finish(hpart, bias_gat, W_lin, b_lin):
    return pl.pallas_call(
        _out_body,
        grid=(N // BN,),
        in_specs=[
            pl.BlockSpec((NC, BN, W), lambda i: (0, i, 0)),
            pl.BlockSpec((1, H), lambda i: (0, 0)),
            pl.BlockSpec((H, O), lambda i: (0, 0)),
            pl.BlockSpec((1, O), lambda i: (0, 0)),
        ],
        out_specs=pl.BlockSpec((BN, O), lambda i: (i, 0)),
        out_shape=jax.ShapeDtypeStruct((N, O), jnp.float32),
    )(hpart, bias_gat.reshape(1, H), W_lin, b_lin.reshape(1, O))


def kernel(x, edge_indices, W_src, W_dst, att_src, att_dst, bias_gat,
           W_lin, b_lin):
    src = edge_indices[0]
    dst = edge_indices[1]
    pad = jnp.zeros((NW, EPT - EPT_REAL), jnp.int32)
    src2 = jnp.concatenate([src.reshape(NW, EPT_REAL), pad],
                           axis=1).reshape(NW * CH, K)
    dst2 = jnp.concatenate([dst.reshape(NW, EPT_REAL), pad],
                           axis=1).reshape(NW * CH, K)

    xsp, a_s2, a_d2 = _project(x, W_src, W_dst, att_src, att_dst)
    e2 = _edge_weights(src2, dst2, a_s2.reshape(N), a_d2.reshape(N))
    hpart = _edge_pass(xsp, src2, dst2, e2)
    return _finish(hpart, bias_gat, W_lin, b_lin)


# speedup vs baseline: 1.0367x; 1.0367x over previous
import jax
import jax.numpy as jnp
from jax import lax
from jax.experimental import pallas as pl
from jax.experimental.pallas import tpu as pltpu
from jax.experimental.pallas import tpu_sc as plsc

N = 10000
D = 128
H = 128
O = 128
E = 320000
W = 144
NC = 2
NS = 16
NW = NC * NS
EPT_REAL = E // NW
K = 64
G = 2
NSETS = 4
SLOTS = NSETS * G
EPT = 10752
CH = EPT // K
BN = 1000
NP = 10112
ROWS_PT = NP // NS

_SC_PARAMS = dict(
    compiler_params=pltpu.CompilerParams(
        needs_layout_passes=False, use_tc_tiling_on_sc=False))


def _proj_body(x_ref, ws_ref, wd_ref, ats_ref, atd_ref,
               xsp_ref, as_ref, ad_ref):
    xb = x_ref[...]
    xs = jnp.dot(xb, ws_ref[...], preferred_element_type=jnp.float32,
                 precision=lax.Precision.HIGHEST)
    xd = jnp.dot(xb, wd_ref[...], preferred_element_type=jnp.float32,
                 precision=lax.Precision.HIGHEST)
    as_ref[...] = jnp.sum(xs * ats_ref[...], axis=1, keepdims=True)
    ad_ref[...] = jnp.sum(xd * atd_ref[...], axis=1, keepdims=True)
    ones = jnp.ones((BN, 1), jnp.float32)
    zeros = jnp.zeros((BN, W - H - 1), jnp.float32)
    xsp_ref[...] = jnp.concatenate([xs, ones, zeros], axis=1)


def _project(x, W_src, W_dst, att_src, att_dst):
    return pl.pallas_call(
        _proj_body,
        grid=(N // BN,),
        in_specs=[
            pl.BlockSpec((BN, D), lambda i: (i, 0)),
            pl.BlockSpec((D, H), lambda i: (0, 0)),
            pl.BlockSpec((D, H), lambda i: (0, 0)),
            pl.BlockSpec((1, H), lambda i: (0, 0)),
            pl.BlockSpec((1, H), lambda i: (0, 0)),
        ],
        out_specs=[
            pl.BlockSpec((BN, W), lambda i: (i, 0)),
            pl.BlockSpec((BN, 1), lambda i: (i, 0)),
            pl.BlockSpec((BN, 1), lambda i: (i, 0)),
        ],
        out_shape=[
            jax.ShapeDtypeStruct((N, W), jnp.float32),
            jax.ShapeDtypeStruct((N, 1), jnp.float32),
            jax.ShapeDtypeStruct((N, 1), jnp.float32),
        ],
    )(x, W_src, W_dst, att_src.reshape(1, H), att_dst.reshape(1, H))


def _weights_body(src_hbm, dst_hbm, as_hbm, ad_hbm, e_hbm,
                  as_v, ad_v, src_sl, dst_sl, e_sl):
    c = lax.axis_index("c")
    s = lax.axis_index("s")
    wid = s * NC + c
    row0 = wid * CH

    pltpu.sync_copy(as_hbm, as_v)
    pltpu.sync_copy(ad_hbm, ad_v)
    pltpu.sync_copy(src_hbm.at[pl.ds(row0, CH)], src_sl)
    pltpu.sync_copy(dst_hbm.at[pl.ds(row0, CH)], dst_sl)

    def _row(r, carry):
        for half in range(K // 16):
            s16 = src_sl[r, pl.ds(half * 16, 16)]
            d16 = dst_sl[r, pl.ds(half * 16, 16)]
            al = plsc.load_gather(as_v, [s16]) + plsc.load_gather(ad_v, [d16])
            al = jnp.where(al >= 0.0, al, al * jnp.float32(0.2))
            ex = jnp.exp(al)
            lid = r * K + half * 16 + lax.iota(jnp.int32, 16)
            e_sl[r, pl.ds(half * 16, 16)] = jnp.where(
                lid < EPT_REAL, ex, jnp.float32(0.0))
        return carry

    lax.fori_loop(0, CH, _row, 0)
    pltpu.sync_copy(e_sl, e_hbm.at[pl.ds(row0, CH)])


def _edge_weights(src2, dst2, a_s, a_d):
    mesh = plsc.VectorSubcoreMesh(core_axis_name="c", subcore_axis_name="s")
    f = pl.kernel(
        _weights_body,
        mesh=mesh,
        out_type=jax.ShapeDtypeStruct((NW * CH, K), jnp.float32),
        scratch_types=[
            pltpu.VMEM((N,), jnp.float32),
            pltpu.VMEM((N,), jnp.float32),
            pltpu.VMEM((CH, K), jnp.int32),
            pltpu.VMEM((CH, K), jnp.int32),
            pltpu.VMEM((CH, K), jnp.float32),
        ],
        **_SC_PARAMS,
    )
    return f(src2, dst2, a_s, a_d)


def _msg_body(xsp_hbm, src_hbm, dst_hbm, e_hbm, out_hbm,
              ss0, ss1, ss2, ss3, ds0, ds1, ds2, ds3,
              es0, es1, es2, es3,
              r0, r1, r2, r3, ebc, h_sh,
              g0, g1, g2, g3, q0, q1, q2, q3, l0, l1, l2, l3):
    srcs = (ss0, ss1, ss2, ss3)
    dsts = (ds0, ds1, ds2, ds3)
    es = (es0, es1, es2, es3)
    rows = (r0, r1, r2, r3)
    gsem = (g0, g1, g2, g3)
    ssem = (q0, q1, q2, q3)
    lsem = (l0, l1, l2, l3)

    c = lax.axis_index("c")
    s = lax.axis_index("s")
    wid = s * NC + c
    row0 = wid * CH

    def _zero_row(k, carry):
        for m in range(W // 16):
            r0[k, pl.ds(m * 16, 16)] = jnp.zeros((16,), jnp.float32)
        return carry
    lax.fori_loop(0, K, _zero_row, 0)
    for i in range(ROWS_PT // K):
        pltpu.sync_copy(r0, h_sh.at[pl.ds(s * ROWS_PT + i * K, K)])
    _rem = ROWS_PT % K
    if _rem:
        pltpu.sync_copy(
            r0.at[pl.ds(0, _rem)],
            h_sh.at[pl.ds(s * ROWS_PT + (ROWS_PT // K) * K, _rem)])
    plsc.subcore_barrier()

    for grp in range(2):
        pltpu.sync_copy(src_hbm.at[pl.ds(row0 + grp * G, G)], srcs[grp])
        pltpu.sync_copy(dst_hbm.at[pl.ds(row0 + grp * G, G)], dsts[grp])
        pltpu.sync_copy(e_hbm.at[pl.ds(row0 + grp * G, G)], es[grp])
    pltpu.async_copy(xsp_hbm.at[ss0.at[0]], r0, g0)
    pltpu.async_copy(xsp_hbm.at[ss0.at[1]], r1, g1)

    def _outer(i, carry):
        for b in range(SLOTS):
            ci = i * SLOTS + b
            p = b % 4
            set_c = b // G
            rc = b % G
            if b % G == 0:
                set_s = (set_c + 2) % NSETS
                @pl.when(ci + 2 * G < CH)
                def _():
                    gro = row0 + ci + 2 * G
                    pltpu.async_copy(src_hbm.at[pl.ds(gro, G)],
                                     srcs[set_s], lsem[set_s])
                    pltpu.async_copy(dst_hbm.at[pl.ds(gro, G)],
                                     dsts[set_s], lsem[set_s])
                    pltpu.async_copy(e_hbm.at[pl.ds(gro, G)],
                                     es[set_s], lsem[set_s])
                set_w = ((b + 2) // G) % NSETS
                @pl.when((ci >= 2) & (ci + 2 < CH))
                def _():
                    gro = row0 + ci + 2
                    pltpu.make_async_copy(src_hbm.at[pl.ds(gro, G)],
                                          srcs[set_w], lsem[set_w]).wait()
                    pltpu.make_async_copy(dst_hbm.at[pl.ds(gro, G)],
                                          dsts[set_w], lsem[set_w]).wait()
                    pltpu.make_async_copy(e_hbm.at[pl.ds(gro, G)],
                                          es[set_w], lsem[set_w]).wait()
            pb = (b - 2) % SLOTS
            set_d, row_d, q = (pb // G) % NSETS, pb % G, pb % 4
            @pl.when(ci >= 2)
            def _():
                pltpu.make_async_copy(
                    rows[q], h_sh.at[dsts[set_d].at[row_d]], ssem[q]).wait()
            nb = b + 2
            set_l, row_l, ql = (nb // G) % NSETS, nb % G, nb % 4
            @pl.when(ci + 2 < CH)
            def _():
                pltpu.async_copy(xsp_hbm.at[srcs[set_l].at[row_l]],
                                 rows[ql], gsem[ql])
            for j in range(K // 16):
                e16 = es[set_c][rc, pl.ds(j * 16, 16)]
                for t in range(16):
                    ebc[j * 16 + t, pl.ds(0, 16)] = jnp.full(
                        (16,), e16[t], jnp.float32)
            pltpu.make_async_copy(xsp_hbm.at[srcs[set_c].at[rc]],
                                  rows[p], gsem[p]).wait()
            def _scale(kk, carry2, _ro=rows[p]):
                for u in range(8):
                    k = kk * 8 + u
                    ek = ebc[k, pl.ds(0, 16)]
                    for m in range(W // 16):
                        _ro[k, pl.ds(m * 16, 16)] = (
                            _ro[k, pl.ds(m * 16, 16)] * ek)
                return carry2
            lax.fori_loop(0, K // 8, _scale, 0)
            pltpu.async_copy(rows[p], h_sh.at[dsts[set_c].at[rc]],
                             ssem[p], add=True)
        return carry

    lax.fori_loop(0, CH // SLOTS, _outer, 0)

    pltpu.make_async_copy(
        rows[(CH - 2) % 4],
        h_sh.at[dsts[((CH - 2) // G) % NSETS].at[(CH - 2) % G]],
        ssem[(CH - 2) % 4]).wait()
    pltpu.make_async_copy(
        rows[(CH - 1) % 4],
        h_sh.at[dsts[((CH - 1) // G) % NSETS].at[(CH - 1) % G]],
        ssem[(CH - 1) % 4]).wait()

    plsc.subcore_barrier()
    for i in range(ROWS_PT // K):
        pltpu.sync_copy(h_sh.at[pl.ds(s * ROWS_PT + i * K, K)],
                        out_hbm.at[c, pl.ds(s * ROWS_PT + i * K, K)])
    if _rem:
        off_r = s * ROWS_PT + (ROWS_PT // K) * K
        pltpu.sync_copy(h_sh.at[pl.ds(off_r, _rem)],
                        out_hbm.at[c, pl.ds(off_r, _rem)])


def _edge_pass(xsp, src2, dst2, e2):
    mesh = plsc.VectorSubcoreMesh(core_axis_name="c", subcore_axis_name="s")
    f = pl.kernel(
        _msg_body,
        mesh=mesh,
        out_type=jax.ShapeDtypeStruct((NC, NP, W), jnp.float32),
        scratch_types=(
            [pltpu.VMEM((G, K), jnp.int32)] * 4
            + [pltpu.VMEM((G, K), jnp.int32)] * 4
            + [pltpu.VMEM((G, K), jnp.float32)] * 4
            + [pltpu.VMEM((K, W), jnp.float32)] * 4
            + [pltpu.VMEM((K, 16), jnp.float32)]
            + [pltpu.VMEM_SHARED((NP, W), jnp.float32)]
            + [pltpu.SemaphoreType.DMA] * 12
        ),
        **_SC_PARAMS,
    )
    return f(xsp, src2, dst2, e2)


def _out_body(hp_ref, bias_ref, wl_ref, bl_ref, o_ref):
    num = hp_ref[0, :, 0:H] + hp_ref[1, :, 0:H]
    den = hp_ref[0, :, H:H + 1] + hp_ref[1, :, H:H + 1]
    h = num / (den + jnp.float32(1e-16)) + bias_ref[...]
    h = jnp.maximum(h, 0.0)
    o_ref[...] = jnp.dot(h, wl_ref[...], preferred_element_type=jnp.float32,
                         precision=lax.Precision.HIGHEST) + bl_ref[...]


def _finish(hpart, bias_gat, W_lin, b_lin):
    return pl.pallas_call(
        _out_body,
        grid=(N // BN,),
        in_specs=[
            pl.BlockSpec((NC, BN, W), lambda i: (0, i, 0)),
            pl.BlockSpec((1, H), lambda i: (0, 0)),
            pl.BlockSpec((H, O), lambda i: (0, 0)),
            pl.BlockSpec((1, O), lambda i: (0, 0)),
        ],
        out_specs=pl.BlockSpec((BN, O), lambda i: (i, 0)),
        out_shape=jax.ShapeDtypeStruct((N, O), jnp.float32),
    )(hpart, bias_gat.reshape(1, H), W_lin, b_lin.reshape(1, O))


def kernel(x, edge_indices, W_src, W_dst, att_src, att_dst, bias_gat,
           W_lin, b_lin):
    src = edge_indices[0]
    dst = edge_indices[1]
    pad = jnp.zeros((NW, EPT - EPT_REAL), jnp.int32)
    src2 = jnp.concatenate([src.reshape(NW, EPT_REAL), pad],
                           axis=1).reshape(NW * CH, K)
    dst2 = jnp.concatenate([dst.reshape(NW, EPT_REAL), pad],
                           axis=1).reshape(NW * CH, K)

    xsp, a_s2, a_d2 = _project(x, W_src, W_dst, att_src, att_dst)
    e2 = _edge_weights(src2, dst2, a_s2.reshape(N), a_d2.reshape(N))
    hpart = _edge_pass(xsp, src2, dst2, e2)
    return _finish(hpart, bias_gat, W_lin, b_lin)


# speedup vs baseline: 2.0625x; 1.9894x over previous
import jax
import jax.numpy as jnp
from jax import lax
from jax.experimental import pallas as pl
from jax.experimental.pallas import tpu as pltpu
from jax.experimental.pallas import tpu_sc as plsc

N = 10000
D = 128
H = 128
O = 128
E = 320000
W = 144
NC = 2
NS = 16
NW = NC * NS
EPT_REAL = E // NW
K = 32
G = 2
NSETS = 4
SLOTS = NSETS * G
EPT = 10240
CH = EPT // K
BN = 1000
NP = 10112
ROWS_PT = NP // NS

_SC_PARAMS = dict(
    compiler_params=pltpu.CompilerParams(
        needs_layout_passes=False, use_tc_tiling_on_sc=False))


def _proj_body(x_ref, ws_ref, wd_ref, ats_ref, atd_ref,
               xsp_ref, as_ref, ad_ref):
    xb = x_ref[...]
    xs = jnp.dot(xb, ws_ref[...], preferred_element_type=jnp.float32,
                 precision=lax.Precision.HIGHEST)
    xd = jnp.dot(xb, wd_ref[...], preferred_element_type=jnp.float32,
                 precision=lax.Precision.HIGHEST)
    as_ref[...] = jnp.sum(xs * ats_ref[...], axis=1, keepdims=True)
    ad_ref[...] = jnp.sum(xd * atd_ref[...], axis=1, keepdims=True)
    ones = jnp.ones((BN, 1), jnp.float32)
    zeros = jnp.zeros((BN, W - H - 1), jnp.float32)
    xsp_ref[...] = jnp.concatenate([xs, ones, zeros], axis=1)


def _project(x, W_src, W_dst, att_src, att_dst):
    return pl.pallas_call(
        _proj_body,
        grid=(N // BN,),
        in_specs=[
            pl.BlockSpec((BN, D), lambda i: (i, 0)),
            pl.BlockSpec((D, H), lambda i: (0, 0)),
            pl.BlockSpec((D, H), lambda i: (0, 0)),
            pl.BlockSpec((1, H), lambda i: (0, 0)),
            pl.BlockSpec((1, H), lambda i: (0, 0)),
        ],
        out_specs=[
            pl.BlockSpec((BN, W), lambda i: (i, 0)),
            pl.BlockSpec((BN, 1), lambda i: (i, 0)),
            pl.BlockSpec((BN, 1), lambda i: (i, 0)),
        ],
        out_shape=[
            jax.ShapeDtypeStruct((N, W), jnp.float32),
            jax.ShapeDtypeStruct((N, 1), jnp.float32),
            jax.ShapeDtypeStruct((N, 1), jnp.float32),
        ],
    )(x, W_src, W_dst, att_src.reshape(1, H), att_dst.reshape(1, H))


def _weights_body(src_hbm, dst_hbm, as_hbm, ad_hbm, e_hbm,
                  as_v, ad_v, src_sl, dst_sl, e_sl):
    c = lax.axis_index("c")
    s = lax.axis_index("s")
    wid = s * NC + c
    row0 = wid * CH

    pltpu.sync_copy(as_hbm, as_v)
    pltpu.sync_copy(ad_hbm, ad_v)
    pltpu.sync_copy(src_hbm.at[pl.ds(row0, CH)], src_sl)
    pltpu.sync_copy(dst_hbm.at[pl.ds(row0, CH)], dst_sl)

    def _row(r, carry):
        for half in range(K // 16):
            s16 = src_sl[r, pl.ds(half * 16, 16)]
            d16 = dst_sl[r, pl.ds(half * 16, 16)]
            al = plsc.load_gather(as_v, [s16]) + plsc.load_gather(ad_v, [d16])
            al = jnp.where(al >= 0.0, al, al * jnp.float32(0.2))
            ex = jnp.exp(al)
            lid = r * K + half * 16 + lax.iota(jnp.int32, 16)
            e_sl[r, pl.ds(half * 16, 16)] = jnp.where(
                lid < EPT_REAL, ex, jnp.float32(0.0))
        return carry

    lax.fori_loop(0, CH, _row, 0)
    pltpu.sync_copy(e_sl, e_hbm.at[pl.ds(row0, CH)])


def _edge_weights(src2, dst2, a_s, a_d):
    mesh = plsc.VectorSubcoreMesh(core_axis_name="c", subcore_axis_name="s")
    f = pl.kernel(
        _weights_body,
        mesh=mesh,
        out_type=jax.ShapeDtypeStruct((NW * CH, K), jnp.float32),
        scratch_types=[
            pltpu.VMEM((N,), jnp.float32),
            pltpu.VMEM((N,), jnp.float32),
            pltpu.VMEM((CH, K), jnp.int32),
            pltpu.VMEM((CH, K), jnp.int32),
            pltpu.VMEM((CH, K), jnp.float32),
        ],
        **_SC_PARAMS,
    )
    return f(src2, dst2, a_s, a_d)


def _msg_body(xsp_hbm, src_hbm, dst_hbm, e_hbm, out_hbm,
              ss0, ss1, ss2, ss3, ds0, ds1, ds2, ds3,
              es0, es1, es2, es3,
              r0, r1, r2, r3, ebc, h_sh,
              g0, g1, g2, g3, q0, q1, q2, q3, l0, l1, l2, l3):
    srcs = (ss0, ss1, ss2, ss3)
    dsts = (ds0, ds1, ds2, ds3)
    es = (es0, es1, es2, es3)
    rows = (r0, r1, r2, r3)
    gsem = (g0, g1, g2, g3)
    ssem = (q0, q1, q2, q3)
    lsem = (l0, l1, l2, l3)

    c = lax.axis_index("c")
    s = lax.axis_index("s")
    wid = s * NC + c
    row0 = wid * CH

    def _zero_row(k, carry):
        for m in range(W // 16):
            r0[k, pl.ds(m * 16, 16)] = jnp.zeros((16,), jnp.float32)
        return carry
    lax.fori_loop(0, K, _zero_row, 0)
    for i in range(ROWS_PT // K):
        pltpu.sync_copy(r0, h_sh.at[pl.ds(s * ROWS_PT + i * K, K)])
    _rem = ROWS_PT % K
    if _rem:
        pltpu.sync_copy(
            r0.at[pl.ds(0, _rem)],
            h_sh.at[pl.ds(s * ROWS_PT + (ROWS_PT // K) * K, _rem)])
    plsc.subcore_barrier()

    for grp in range(2):
        pltpu.sync_copy(src_hbm.at[pl.ds(row0 + grp * G, G)], srcs[grp])
        pltpu.sync_copy(dst_hbm.at[pl.ds(row0 + grp * G, G)], dsts[grp])
        pltpu.sync_copy(e_hbm.at[pl.ds(row0 + grp * G, G)], es[grp])
    pltpu.async_copy(xsp_hbm.at[ss0.at[0]], r0, g0)
    pltpu.async_copy(xsp_hbm.at[ss0.at[1]], r1, g1)

    def _outer(i, carry):
        for b in range(SLOTS):
            ci = i * SLOTS + b
            p = b % 4
            set_c = b // G
            rc = b % G
            if b % G == 0:
                set_s = (set_c + 2) % NSETS
                @pl.when(ci + 2 * G < CH)
                def _():
                    gro = row0 + ci + 2 * G
                    pltpu.async_copy(src_hbm.at[pl.ds(gro, G)],
                                     srcs[set_s], lsem[set_s])
                    pltpu.async_copy(dst_hbm.at[pl.ds(gro, G)],
                                     dsts[set_s], lsem[set_s])
                    pltpu.async_copy(e_hbm.at[pl.ds(gro, G)],
                                     es[set_s], lsem[set_s])
                set_w = ((b + 2) // G) % NSETS
                @pl.when((ci >= 2) & (ci + 2 < CH))
                def _():
                    gro = row0 + ci + 2
                    pltpu.make_async_copy(src_hbm.at[pl.ds(gro, G)],
                                          srcs[set_w], lsem[set_w]).wait()
                    pltpu.make_async_copy(dst_hbm.at[pl.ds(gro, G)],
                                          dsts[set_w], lsem[set_w]).wait()
                    pltpu.make_async_copy(e_hbm.at[pl.ds(gro, G)],
                                          es[set_w], lsem[set_w]).wait()
            pb = (b - 2) % SLOTS
            set_d, row_d, q = (pb // G) % NSETS, pb % G, pb % 4
            @pl.when(ci >= 2)
            def _():
                pltpu.make_async_copy(
                    rows[q], h_sh.at[dsts[set_d].at[row_d]], ssem[q]).wait()
            nb = b + 2
            set_l, row_l, ql = (nb // G) % NSETS, nb % G, nb % 4
            @pl.when(ci + 2 < CH)
            def _():
                pltpu.async_copy(xsp_hbm.at[srcs[set_l].at[row_l]],
                                 rows[ql], gsem[ql])
            for j in range(K // 16):
                e16 = es[set_c][rc, pl.ds(j * 16, 16)]
                for t in range(16):
                    ebc[j * 16 + t, pl.ds(0, 16)] = jnp.full(
                        (16,), e16[t], jnp.float32)
            pltpu.make_async_copy(xsp_hbm.at[srcs[set_c].at[rc]],
                                  rows[p], gsem[p]).wait()
            def _scale(kk, carry2, _ro=rows[p]):
                for u in range(8):
                    k = kk * 8 + u
                    ek = ebc[k, pl.ds(0, 16)]
                    for m in range(W // 16):
                        _ro[k, pl.ds(m * 16, 16)] = (
                            _ro[k, pl.ds(m * 16, 16)] * ek)
                return carry2
            lax.fori_loop(0, K // 8, _scale, 0)
            pltpu.async_copy(rows[p], h_sh.at[dsts[set_c].at[rc]],
                             ssem[p], add=True)
        return carry

    lax.fori_loop(0, CH // SLOTS, _outer, 0)

    pltpu.make_async_copy(
        rows[(CH - 2) % 4],
        h_sh.at[dsts[((CH - 2) // G) % NSETS].at[(CH - 2) % G]],
        ssem[(CH - 2) % 4]).wait()
    pltpu.make_async_copy(
        rows[(CH - 1) % 4],
        h_sh.at[dsts[((CH - 1) // G) % NSETS].at[(CH - 1) % G]],
        ssem[(CH - 1) % 4]).wait()

    plsc.subcore_barrier()
    for i in range(ROWS_PT // K):
        pltpu.sync_copy(h_sh.at[pl.ds(s * ROWS_PT + i * K, K)],
                        out_hbm.at[c, pl.ds(s * ROWS_PT + i * K, K)])
    if _rem:
        off_r = s * ROWS_PT + (ROWS_PT // K) * K
        pltpu.sync_copy(h_sh.at[pl.ds(off_r, _rem)],
                        out_hbm.at[c, pl.ds(off_r, _rem)])


def _edge_pass(xsp, src2, dst2, e2):
    mesh = plsc.VectorSubcoreMesh(core_axis_name="c", subcore_axis_name="s")
    f = pl.kernel(
        _msg_body,
        mesh=mesh,
        out_type=jax.ShapeDtypeStruct((NC, NP, W), jnp.float32),
        scratch_types=(
            [pltpu.VMEM((G, K), jnp.int32)] * 4
            + [pltpu.VMEM((G, K), jnp.int32)] * 4
            + [pltpu.VMEM((G, K), jnp.float32)] * 4
            + [pltpu.VMEM((K, W), jnp.float32)] * 4
            + [pltpu.VMEM((K, 16), jnp.float32)]
            + [pltpu.VMEM_SHARED((NP, W), jnp.float32)]
            + [pltpu.SemaphoreType.DMA] * 12
        ),
        **_SC_PARAMS,
    )
    return f(xsp, src2, dst2, e2)


def _out_body(hp_ref, bias_ref, wl_ref, bl_ref, o_ref):
    num = hp_ref[0, :, 0:H] + hp_ref[1, :, 0:H]
    den = hp_ref[0, :, H:H + 1] + hp_ref[1, :, H:H + 1]
    h = num / (den + jnp.float32(1e-16)) + bias_ref[...]
    h = jnp.maximum(h, 0.0)
    o_ref[...] = jnp.dot(h, wl_ref[...], preferred_element_type=jnp.float32,
                         precision=lax.Precision.HIGHEST) + bl_ref[...]


def _finish(hpart, bias_gat, W_lin, b_lin):
    return pl.pallas_call(
        _out_body,
        grid=(N // BN,),
        in_specs=[
            pl.BlockSpec((NC, BN, W), lambda i: (0, i, 0)),
            pl.BlockSpec((1, H), lambda i: (0, 0)),
            pl.BlockSpec((H, O), lambda i: (0, 0)),
            pl.BlockSpec((1, O), lambda i: (0, 0)),
        ],
        out_specs=pl.BlockSpec((BN, O), lambda i: (i, 0)),
        out_shape=jax.ShapeDtypeStruct((N, O), jnp.float32),
    )(hpart, bias_gat.reshape(1, H), W_lin, b_lin.reshape(1, O))


def kernel(x, edge_indices, W_src, W_dst, att_src, att_dst, bias_gat,
           W_lin, b_lin):
    src = edge_indices[0]
    dst = edge_indices[1]
    pad = jnp.zeros((NW, EPT - EPT_REAL), jnp.int32)
    src2 = jnp.concatenate([src.reshape(NW, EPT_REAL), pad],
                           axis=1).reshape(NW * CH, K)
    dst2 = jnp.concatenate([dst.reshape(NW, EPT_REAL), pad],
                           axis=1).reshape(NW * CH, K)

    xsp, a_s2, a_d2 = _project(x, W_src, W_dst, att_src, att_dst)
    e2 = _edge_weights(src2, dst2, a_s2.reshape(N), a_d2.reshape(N))
    hpart = _edge_pass(xsp, src2, dst2, e2)
    return _finish(hpart, bias_gat, W_lin, b_lin)
